# baseline (speedup 1.0000x reference)
import functools

import numpy as np
import jax
import jax.numpy as jnp
from jax import lax
from jax.experimental import pallas as pl
from jax.experimental.pallas import tpu as pltpu
from jax.experimental.pallas import tpu_sc as plsc

POS_RADIUS = 0.1
NEG_RADIUS = 0.2
TRIPLET_MARGIN = 1.0
MAX_ANCHOR = 1024
MAX_DST = 8192
VOTING_START = 0

_N = 20000
_rng = np.random.RandomState(0)
_PERM_SRC = np.ascontiguousarray(_rng.permutation(_N)[:MAX_ANCHOR].astype(np.int32))
_PERM_DST = np.ascontiguousarray(_rng.permutation(_N)[:MAX_DST].astype(np.int32))

_NN_BLK = 2000
_NEG_BLK = 2048

_NW = 32
_BS = MAX_ANCHOR // _NW
_BD = MAX_DST // _NW
_BS3 = _BS * 3
_BD3 = _BD * 3
_IDXP_SRC = np.ascontiguousarray(
    (_PERM_SRC[None, :] * 3 + np.arange(3)[:, None]).reshape(-1).astype(np.int32))
_IDXP_DST = np.ascontiguousarray(
    (_PERM_DST[None, :] * 3 + np.arange(3)[:, None]).reshape(-1).astype(np.int32))

_SC_MESH = dict(core_axis_name="c", subcore_axis_name="s")
_SC_PARAMS = dict(
    mesh=plsc.VectorSubcoreMesh(**_SC_MESH),
    compiler_params=pltpu.CompilerParams(use_tc_tiling_on_sc=False),
)


def _sc_gather_srcside(src_flat, src_desc, src_scores):
    psp = jnp.asarray(_IDXP_SRC)
    ps = jnp.asarray(_PERM_SRC)

    @functools.partial(
        pl.kernel,
        out_type=[
            jax.ShapeDtypeStruct((3 * MAX_ANCHOR,), jnp.float32),
            jax.ShapeDtypeStruct((MAX_ANCHOR, 64), jnp.float32),
            jax.ShapeDtypeStruct((MAX_ANCHOR,), jnp.float32),
        ],
        scratch_types=[
            pltpu.VMEM((_BS3,), jnp.int32),
            pltpu.VMEM((_BS3,), jnp.float32),
            pltpu.VMEM((_BS,), jnp.int32),
            pltpu.VMEM((_BS, 64), jnp.float32),
            pltpu.VMEM((_BS,), jnp.float32),
            pltpu.SemaphoreType.DMA,
        ],
        **_SC_PARAMS,
    )
    def k(sflat, sdesc, sscore, psp_h, ps_h, o_pcs, o_anc, o_ss,
          ipsp, b_pcs, ips, b_anc, b_ss, sem):
        wid = lax.axis_index("s") * 2 + lax.axis_index("c")
        b3 = wid * _BS3
        b1 = wid * _BS
        pltpu.sync_copy(psp_h.at[pl.ds(b3, _BS3)], ipsp)
        pltpu.sync_copy(ps_h.at[pl.ds(b1, _BS)], ips)
        pltpu.async_copy(sflat.at[ipsp], b_pcs, sem).wait()
        pltpu.sync_copy(b_pcs, o_pcs.at[pl.ds(b3, _BS3)])
        pltpu.async_copy(sdesc.at[ips], b_anc, sem).wait()
        pltpu.sync_copy(b_anc, o_anc.at[pl.ds(b1, _BS)])
        pltpu.async_copy(sscore.at[ips], b_ss, sem).wait()
        pltpu.sync_copy(b_ss, o_ss.at[pl.ds(b1, _BS)])

    pcs_f, anc, ss = k(src_flat, src_desc, src_scores, psp, ps)
    return jnp.reshape(pcs_f, (3, MAX_ANCHOR)), anc, ss


def _sc_gather_dstside(dst_flat, dst_desc):
    pdp = jnp.asarray(_IDXP_DST)
    pd = jnp.asarray(_PERM_DST)

    @functools.partial(
        pl.kernel,
        out_type=[
            jax.ShapeDtypeStruct((3 * MAX_DST,), jnp.float32),
            jax.ShapeDtypeStruct((MAX_DST, 64), jnp.float32),
        ],
        scratch_types=[
            pltpu.VMEM((_BD3,), jnp.int32),
            pltpu.VMEM((_BD3,), jnp.float32),
            pltpu.VMEM((_BD,), jnp.int32),
            pltpu.VMEM((_BD, 64), jnp.float32),
            pltpu.SemaphoreType.DMA,
        ],
        **_SC_PARAMS,
    )
    def k(dflat, ddesc, pdp_h, pd_h, o_pcd, o_dds,
          ipdp, b_pcd, ipd, b_dds, sem):
        wid = lax.axis_index("s") * 2 + lax.axis_index("c")
        b3 = wid * _BD3
        b1 = wid * _BD
        pltpu.sync_copy(pdp_h.at[pl.ds(b3, _BD3)], ipdp)
        pltpu.sync_copy(pd_h.at[pl.ds(b1, _BD)], ipd)
        pltpu.async_copy(dflat.at[ipdp], b_pcd, sem).wait()
        pltpu.sync_copy(b_pcd, o_pcd.at[pl.ds(b3, _BD3)])
        pltpu.async_copy(ddesc.at[ipd], b_dds, sem).wait()
        pltpu.sync_copy(b_dds, o_dds.at[pl.ds(b1, _BD)])

    pcd_f, dds = k(dst_flat, dst_desc, pdp, pd)
    return jnp.reshape(pcd_f, (3, MAX_DST)), dds


def _sc_gather_nnside(dst_desc, dst_scores, nn):
    @functools.partial(
        pl.kernel,
        out_type=[
            jax.ShapeDtypeStruct((MAX_ANCHOR, 64), jnp.float32),
            jax.ShapeDtypeStruct((MAX_ANCHOR,), jnp.float32),
        ],
        scratch_types=[
            pltpu.VMEM((_BS,), jnp.int32),
            pltpu.VMEM((_BS, 64), jnp.float32),
            pltpu.VMEM((_BS,), jnp.float32),
            pltpu.SemaphoreType.DMA,
        ],
        **_SC_PARAMS,
    )
    def k(ddesc, dscore, nn_h, o_pos, o_ns, inn, b_pos, b_ns, sem):
        wid = lax.axis_index("s") * 2 + lax.axis_index("c")
        b1 = wid * _BS
        pltpu.sync_copy(nn_h.at[pl.ds(b1, _BS)], inn)
        pltpu.async_copy(ddesc.at[inn], b_pos, sem).wait()
        pltpu.sync_copy(b_pos, o_pos.at[pl.ds(b1, _BS)])
        pltpu.async_copy(dscore.at[inn], b_ns, sem).wait()
        pltpu.sync_copy(b_ns, o_ns.at[pl.ds(b1, _BS)])

    return k(dst_desc, dst_scores, nn)


def _nn_kernel(a_ref, b_ref, mind_ref, idx_ref):
    j = pl.program_id(0)
    a = a_ref[...]
    b = b_ref[...]
    sa = jnp.sum(a * a, axis=0)
    sb = jnp.sum(b * b, axis=1)
    prod = lax.dot_general(a, b, (((0,), (1,)), ((), ())),
                           preferred_element_type=jnp.float32)
    d2 = (sa[:, None] + sb[None, :]) - 2.0 * prod
    d = jnp.sqrt(jnp.maximum(d2, 1e-12))
    col = j * _NN_BLK + lax.broadcasted_iota(jnp.int32, d.shape, 1)
    blk_min = jnp.min(d, axis=1)
    blk_idx = jnp.min(jnp.where(d == blk_min[:, None], col, _N), axis=1)

    @pl.when(j == 0)
    def _():
        mind_ref[...] = blk_min
        idx_ref[...] = blk_idx

    @pl.when(j > 0)
    def _():
        prev = mind_ref[...]
        better = blk_min < prev
        mind_ref[...] = jnp.where(better, blk_min, prev)
        idx_ref[...] = jnp.where(better, blk_idx, idx_ref[...])


def _nn_search(pc_srcT, dst_xyz):
    grid = _N // _NN_BLK
    mind, idx = pl.pallas_call(
        _nn_kernel,
        grid=(grid,),
        in_specs=[
            pl.BlockSpec((3, MAX_ANCHOR), lambda j: (0, 0)),
            pl.BlockSpec((_NN_BLK, 3), lambda j: (j, 0)),
        ],
        out_specs=[
            pl.BlockSpec((MAX_ANCHOR,), lambda j: (0,)),
            pl.BlockSpec((MAX_ANCHOR,), lambda j: (0,)),
        ],
        out_shape=[
            jax.ShapeDtypeStruct((MAX_ANCHOR,), jnp.float32),
            jax.ShapeDtypeStruct((MAX_ANCHOR,), jnp.int32),
        ],
    )(pc_srcT, dst_xyz)
    return mind, idx


def _negmin_kernel(a_ref, ad_ref, b_ref, bd_ref, negmin_ref):
    j = pl.program_id(0)
    a = a_ref[...]
    b = b_ref[...]
    ad = ad_ref[...]
    bd = bd_ref[...]
    sa = jnp.sum(a * a, axis=0)
    sad = jnp.sum(ad * ad, axis=1)
    sb = jnp.sum(b * b, axis=0)
    sbd = jnp.sum(bd * bd, axis=1)

    prod_x = lax.dot_general(a, b, (((0,), (0,)), ((), ())),
                             preferred_element_type=jnp.float32)
    dist2 = (sa[:, None] + sb[None, :]) - 2.0 * prod_x
    dist = jnp.sqrt(jnp.maximum(dist2, 1e-12))

    prod_d = lax.dot_general(ad, bd, (((1,), (1,)), ((), ())),
                             preferred_element_type=jnp.float32)
    desc2 = (sad[:, None] + sbd[None, :]) - 2.0 * prod_d
    desc = jnp.sqrt(jnp.maximum(desc2, 1e-12))
    desc = desc + jnp.where(dist < NEG_RADIUS, 1e10, 0.0)
    blk_min = jnp.min(desc, axis=1)

    @pl.when(j == 0)
    def _():
        negmin_ref[...] = blk_min

    @pl.when(j > 0)
    def _():
        negmin_ref[...] = jnp.minimum(negmin_ref[...], blk_min)


def _negmin(pc_srcT, anc_desc, pc_dstT, desc_dst_sub):
    grid = MAX_DST // _NEG_BLK
    return pl.pallas_call(
        _negmin_kernel,
        grid=(grid,),
        in_specs=[
            pl.BlockSpec((3, MAX_ANCHOR), lambda j: (0, 0)),
            pl.BlockSpec((MAX_ANCHOR, 64), lambda j: (0, 0)),
            pl.BlockSpec((3, _NEG_BLK), lambda j: (0, j)),
            pl.BlockSpec((_NEG_BLK, 64), lambda j: (j, 0)),
        ],
        out_specs=pl.BlockSpec((MAX_ANCHOR,), lambda j: (0,)),
        out_shape=jax.ShapeDtypeStruct((MAX_ANCHOR,), jnp.float32),
    )(pc_srcT, anc_desc, pc_dstT, desc_dst_sub)


def _final_kernel(negmin_ref, ad_ref, pos_ref, ss_ref, ns_ref, nnd_ref,
                  out_ref):
    negative_min = negmin_ref[...]
    ad = ad_ref[...]
    pos = pos_ref[...]
    diff = ad - pos
    positive_max = jnp.sqrt(jnp.sum(diff * diff, axis=1) + 1e-12)
    p_n_diff = positive_max - negative_min
    nnd = nnd_ref[...]
    maskf = (nnd < POS_RADIUS).astype(jnp.float32)
    count = jnp.sum(maskf)
    sel_sigma = (ss_ref[...] + ns_ref[...]) * 0.5
    desc_loss = jnp.sum(jnp.maximum(p_n_diff + TRIPLET_MARGIN, 0.0) * maskf)
    score_loss = jnp.sum(sel_sigma * p_n_diff * maskf)
    loss = (desc_loss + score_loss) / count
    loss = jnp.where(count < float(MAX_ANCHOR // 2), 0.0, loss)
    out_ref[...] = loss.reshape(1, 1)


def _final_loss(negmin, anc_desc, pos_desc, s_src, s_nn, nn_d):
    out = pl.pallas_call(
        _final_kernel,
        out_shape=jax.ShapeDtypeStruct((1, 1), jnp.float32),
    )(negmin, anc_desc, pos_desc, s_src, s_nn, nn_d)
    return out[0, 0]


def kernel(src_xyz, src_desc, src_scores, dst_xyz, dst_desc, dst_scores, epoch):
    src_flat = jnp.reshape(src_xyz, (-1,))
    dst_flat = jnp.reshape(dst_xyz, (-1,))
    pc_srcT, anc_desc, s_src = _sc_gather_srcside(src_flat, src_desc, src_scores)
    pc_dstT, desc_dst_sub = _sc_gather_dstside(dst_flat, dst_desc)

    nn_d, nn = _nn_search(pc_srcT, dst_xyz)

    pos_desc, s_nn = _sc_gather_nnside(dst_desc, dst_scores, nn)

    negmin = _negmin(pc_srcT, anc_desc, pc_dstT, desc_dst_sub)
    loss = _final_loss(negmin, anc_desc, pos_desc, s_src, s_nn, nn_d)
    out = jnp.where(jnp.asarray(epoch) <= VOTING_START, 0.0, loss)
    return out.astype(jnp.float32)


# speedup vs baseline: 1.0116x; 1.0116x over previous
import functools

import numpy as np
import jax
import jax.numpy as jnp
from jax import lax
from jax.experimental import pallas as pl
from jax.experimental.pallas import tpu as pltpu
from jax.experimental.pallas import tpu_sc as plsc

POS_RADIUS = 0.1
NEG_RADIUS = 0.2
TRIPLET_MARGIN = 1.0
MAX_ANCHOR = 1024
MAX_DST = 8192
VOTING_START = 0

_N = 20000
_rng = np.random.RandomState(0)
_PERM_SRC = np.ascontiguousarray(_rng.permutation(_N)[:MAX_ANCHOR].astype(np.int32))
_PERM_DST = np.ascontiguousarray(_rng.permutation(_N)[:MAX_DST].astype(np.int32))

_NN_BLK = 2000
_NEG_BLK = 2048

_NW = 32
_BS = MAX_ANCHOR // _NW
_BD = MAX_DST // _NW
_BS3 = _BS * 3
_BD3 = _BD * 3
_IDXP_SRC = np.ascontiguousarray(
    (_PERM_SRC[None, :] * 3 + np.arange(3)[:, None]).reshape(-1).astype(np.int32))
_IDXP_DST = np.ascontiguousarray(
    (_PERM_DST[None, :] * 3 + np.arange(3)[:, None]).reshape(-1).astype(np.int32))

_SC_MESH = dict(core_axis_name="c", subcore_axis_name="s")
_SC_PARAMS = dict(
    mesh=plsc.VectorSubcoreMesh(**_SC_MESH),
    compiler_params=pltpu.CompilerParams(use_tc_tiling_on_sc=False),
)


def _sc_gather_srcside(src_flat, src_scores):
    psp = jnp.asarray(_IDXP_SRC)
    ps = jnp.asarray(_PERM_SRC)

    @functools.partial(
        pl.kernel,
        out_type=[
            jax.ShapeDtypeStruct((3 * MAX_ANCHOR,), jnp.float32),
            jax.ShapeDtypeStruct((MAX_ANCHOR,), jnp.float32),
        ],
        scratch_types=[
            pltpu.VMEM((_BS3,), jnp.int32),
            pltpu.VMEM((_BS3,), jnp.float32),
            pltpu.VMEM((_BS,), jnp.int32),
            pltpu.VMEM((_BS,), jnp.float32),
            pltpu.SemaphoreType.DMA,
        ],
        **_SC_PARAMS,
    )
    def k(sflat, sscore, psp_h, ps_h, o_pcs, o_ss,
          ipsp, b_pcs, ips, b_ss, sem):
        wid = lax.axis_index("s") * 2 + lax.axis_index("c")
        b3 = wid * _BS3
        b1 = wid * _BS
        pltpu.sync_copy(psp_h.at[pl.ds(b3, _BS3)], ipsp)
        pltpu.sync_copy(ps_h.at[pl.ds(b1, _BS)], ips)
        pltpu.async_copy(sflat.at[ipsp], b_pcs, sem).wait()
        pltpu.sync_copy(b_pcs, o_pcs.at[pl.ds(b3, _BS3)])
        pltpu.async_copy(sscore.at[ips], b_ss, sem).wait()
        pltpu.sync_copy(b_ss, o_ss.at[pl.ds(b1, _BS)])

    pcs_f, ss = k(src_flat, src_scores, psp, ps)
    return jnp.reshape(pcs_f, (3, MAX_ANCHOR)), ss


def _sc_gather_dstside(dst_flat):
    pdp = jnp.asarray(_IDXP_DST)

    @functools.partial(
        pl.kernel,
        out_type=jax.ShapeDtypeStruct((3 * MAX_DST,), jnp.float32),
        scratch_types=[
            pltpu.VMEM((_BD3,), jnp.int32),
            pltpu.VMEM((_BD3,), jnp.float32),
            pltpu.SemaphoreType.DMA,
        ],
        **_SC_PARAMS,
    )
    def k(dflat, pdp_h, o_pcd, ipdp, b_pcd, sem):
        wid = lax.axis_index("s") * 2 + lax.axis_index("c")
        b3 = wid * _BD3
        pltpu.sync_copy(pdp_h.at[pl.ds(b3, _BD3)], ipdp)
        pltpu.async_copy(dflat.at[ipdp], b_pcd, sem).wait()
        pltpu.sync_copy(b_pcd, o_pcd.at[pl.ds(b3, _BD3)])

    pcd_f = k(dst_flat, pdp)
    return jnp.reshape(pcd_f, (3, MAX_DST))


def _sc_gather_dstdesc(dst_desc, dst_scores, nn):
    pd = jnp.asarray(_PERM_DST)

    @functools.partial(
        pl.kernel,
        out_type=[
            jax.ShapeDtypeStruct((MAX_DST, 64), jnp.float32),
            jax.ShapeDtypeStruct((MAX_ANCHOR, 64), jnp.float32),
            jax.ShapeDtypeStruct((MAX_ANCHOR,), jnp.float32),
        ],
        scratch_types=[
            pltpu.VMEM((_BD,), jnp.int32),
            pltpu.VMEM((_BD, 64), jnp.float32),
            pltpu.VMEM((_BS,), jnp.int32),
            pltpu.VMEM((_BS, 64), jnp.float32),
            pltpu.VMEM((_BS,), jnp.float32),
            pltpu.SemaphoreType.DMA,
        ],
        **_SC_PARAMS,
    )
    def k(ddesc, dscore, pd_h, nn_h, o_dds, o_pos, o_ns,
          ipd, b_dds, inn, b_pos, b_ns, sem):
        wid = lax.axis_index("s") * 2 + lax.axis_index("c")
        b2 = wid * _BD
        b1 = wid * _BS
        pltpu.sync_copy(pd_h.at[pl.ds(b2, _BD)], ipd)
        pltpu.sync_copy(nn_h.at[pl.ds(b1, _BS)], inn)
        pltpu.async_copy(ddesc.at[ipd], b_dds, sem).wait()
        pltpu.sync_copy(b_dds, o_dds.at[pl.ds(b2, _BD)])
        pltpu.async_copy(ddesc.at[inn], b_pos, sem).wait()
        pltpu.sync_copy(b_pos, o_pos.at[pl.ds(b1, _BS)])
        pltpu.async_copy(dscore.at[inn], b_ns, sem).wait()
        pltpu.sync_copy(b_ns, o_ns.at[pl.ds(b1, _BS)])

    return k(dst_desc, dst_scores, pd, nn)


def _nn_kernel(a_ref, b_ref, mind_ref, idx_ref):
    j = pl.program_id(0)
    a = a_ref[...]
    b = b_ref[...]
    sa = jnp.sum(a * a, axis=0)
    sb = jnp.sum(b * b, axis=1)
    prod = lax.dot_general(a, b, (((0,), (1,)), ((), ())),
                           preferred_element_type=jnp.float32)
    d2 = (sa[:, None] + sb[None, :]) - 2.0 * prod
    d = jnp.sqrt(jnp.maximum(d2, 1e-12))
    col = j * _NN_BLK + lax.broadcasted_iota(jnp.int32, d.shape, 1)
    blk_min = jnp.min(d, axis=1)
    blk_idx = jnp.min(jnp.where(d == blk_min[:, None], col, _N), axis=1)

    @pl.when(j == 0)
    def _():
        mind_ref[...] = blk_min
        idx_ref[...] = blk_idx

    @pl.when(j > 0)
    def _():
        prev = mind_ref[...]
        better = blk_min < prev
        mind_ref[...] = jnp.where(better, blk_min, prev)
        idx_ref[...] = jnp.where(better, blk_idx, idx_ref[...])


def _nn_search(pc_srcT, dst_xyz):
    grid = _N // _NN_BLK
    mind, idx = pl.pallas_call(
        _nn_kernel,
        grid=(grid,),
        in_specs=[
            pl.BlockSpec((3, MAX_ANCHOR), lambda j: (0, 0)),
            pl.BlockSpec((_NN_BLK, 3), lambda j: (j, 0)),
        ],
        out_specs=[
            pl.BlockSpec((MAX_ANCHOR,), lambda j: (0,)),
            pl.BlockSpec((MAX_ANCHOR,), lambda j: (0,)),
        ],
        out_shape=[
            jax.ShapeDtypeStruct((MAX_ANCHOR,), jnp.float32),
            jax.ShapeDtypeStruct((MAX_ANCHOR,), jnp.int32),
        ],
    )(pc_srcT, dst_xyz)
    return mind, idx


def _negmin_kernel(a_ref, ad_ref, b_ref, bd_ref, negmin_ref):
    j = pl.program_id(0)
    a = a_ref[...]
    b = b_ref[...]
    ad = ad_ref[...]
    bd = bd_ref[...]
    sa = jnp.sum(a * a, axis=0)
    sad = jnp.sum(ad * ad, axis=1)
    sb = jnp.sum(b * b, axis=0)
    sbd = jnp.sum(bd * bd, axis=1)

    prod_x = lax.dot_general(a, b, (((0,), (0,)), ((), ())),
                             preferred_element_type=jnp.float32)
    dist2 = (sa[:, None] + sb[None, :]) - 2.0 * prod_x
    dist = jnp.sqrt(jnp.maximum(dist2, 1e-12))

    prod_d = lax.dot_general(ad, bd, (((1,), (1,)), ((), ())),
                             preferred_element_type=jnp.float32)
    desc2 = (sad[:, None] + sbd[None, :]) - 2.0 * prod_d
    desc = jnp.sqrt(jnp.maximum(desc2, 1e-12))
    desc = desc + jnp.where(dist < NEG_RADIUS, 1e10, 0.0)
    blk_min = jnp.min(desc, axis=1)

    @pl.when(j == 0)
    def _():
        negmin_ref[...] = blk_min

    @pl.when(j > 0)
    def _():
        negmin_ref[...] = jnp.minimum(negmin_ref[...], blk_min)


def _negmin(pc_srcT, anc_desc, pc_dstT, desc_dst_sub):
    grid = MAX_DST // _NEG_BLK
    return pl.pallas_call(
        _negmin_kernel,
        grid=(grid,),
        in_specs=[
            pl.BlockSpec((3, MAX_ANCHOR), lambda j: (0, 0)),
            pl.BlockSpec((MAX_ANCHOR, 64), lambda j: (0, 0)),
            pl.BlockSpec((3, _NEG_BLK), lambda j: (0, j)),
            pl.BlockSpec((_NEG_BLK, 64), lambda j: (j, 0)),
        ],
        out_specs=pl.BlockSpec((MAX_ANCHOR,), lambda j: (0,)),
        out_shape=jax.ShapeDtypeStruct((MAX_ANCHOR,), jnp.float32),
    )(pc_srcT, anc_desc, pc_dstT, desc_dst_sub)


def _final_kernel(negmin_ref, ad_ref, pos_ref, ss_ref, ns_ref, nnd_ref,
                  out_ref):
    negative_min = negmin_ref[...]
    ad = ad_ref[...]
    pos = pos_ref[...]
    diff = ad - pos
    positive_max = jnp.sqrt(jnp.sum(diff * diff, axis=1) + 1e-12)
    p_n_diff = positive_max - negative_min
    nnd = nnd_ref[...]
    maskf = (nnd < POS_RADIUS).astype(jnp.float32)
    count = jnp.sum(maskf)
    sel_sigma = (ss_ref[...] + ns_ref[...]) * 0.5
    desc_loss = jnp.sum(jnp.maximum(p_n_diff + TRIPLET_MARGIN, 0.0) * maskf)
    score_loss = jnp.sum(sel_sigma * p_n_diff * maskf)
    loss = (desc_loss + score_loss) / count
    loss = jnp.where(count < float(MAX_ANCHOR // 2), 0.0, loss)
    out_ref[...] = loss.reshape(1, 1)


def _final_loss(negmin, anc_desc, pos_desc, s_src, s_nn, nn_d):
    out = pl.pallas_call(
        _final_kernel,
        out_shape=jax.ShapeDtypeStruct((1, 1), jnp.float32),
    )(negmin, anc_desc, pos_desc, s_src, s_nn, nn_d)
    return out[0, 0]


def kernel(src_xyz, src_desc, src_scores, dst_xyz, dst_desc, dst_scores, epoch):
    src_flat = jnp.reshape(src_xyz, (-1,))
    dst_flat = jnp.reshape(dst_xyz, (-1,))
    pc_srcT, s_src = _sc_gather_srcside(src_flat, src_scores)
    anc_desc = src_desc[_PERM_SRC]
    pc_dstT = _sc_gather_dstside(dst_flat)

    nn_d, nn = _nn_search(pc_srcT, dst_xyz)

    desc_dst_sub, pos_desc, s_nn = _sc_gather_dstdesc(dst_desc, dst_scores, nn)

    negmin = _negmin(pc_srcT, anc_desc, pc_dstT, desc_dst_sub)
    loss = _final_loss(negmin, anc_desc, pos_desc, s_src, s_nn, nn_d)
    out = jnp.where(jnp.asarray(epoch) <= VOTING_START, 0.0, loss)
    return out.astype(jnp.float32)


# speedup vs baseline: 1.0351x; 1.0232x over previous
import functools

import numpy as np
import jax
import jax.numpy as jnp
from jax import lax
from jax.experimental import pallas as pl
from jax.experimental.pallas import tpu as pltpu
from jax.experimental.pallas import tpu_sc as plsc

POS_RADIUS = 0.1
NEG_RADIUS = 0.2
TRIPLET_MARGIN = 1.0
MAX_ANCHOR = 1024
MAX_DST = 8192
VOTING_START = 0

_N = 20000
_rng = np.random.RandomState(0)
_PERM_SRC = np.ascontiguousarray(_rng.permutation(_N)[:MAX_ANCHOR].astype(np.int32))
_PERM_DST = np.ascontiguousarray(_rng.permutation(_N)[:MAX_DST].astype(np.int32))

_NN_BLK = 2000
_NEG_BLK = 2048

_NW = 32
_BS = MAX_ANCHOR // _NW
_BD = MAX_DST // _NW
_BS3 = _BS * 3
_BD3 = _BD * 3
_IDXP_SRC = np.ascontiguousarray(
    (_PERM_SRC[None, :] * 3 + np.arange(3)[:, None]).reshape(-1).astype(np.int32))
_IDXP_DST = np.ascontiguousarray(
    (_PERM_DST[None, :] * 3 + np.arange(3)[:, None]).reshape(-1).astype(np.int32))

_SC_MESH = dict(core_axis_name="c", subcore_axis_name="s")
_SC_PARAMS = dict(
    mesh=plsc.VectorSubcoreMesh(**_SC_MESH),
    compiler_params=pltpu.CompilerParams(use_tc_tiling_on_sc=False),
)


def _sc_gather_srcside(src_flat, src_scores):
    psp = jnp.asarray(_IDXP_SRC)
    ps = jnp.asarray(_PERM_SRC)

    @functools.partial(
        pl.kernel,
        out_type=[
            jax.ShapeDtypeStruct((3 * MAX_ANCHOR,), jnp.float32),
            jax.ShapeDtypeStruct((MAX_ANCHOR,), jnp.float32),
        ],
        scratch_types=[
            pltpu.VMEM((_BS3,), jnp.int32),
            pltpu.VMEM((_BS3,), jnp.float32),
            pltpu.VMEM((_BS,), jnp.int32),
            pltpu.VMEM((_BS,), jnp.float32),
            pltpu.SemaphoreType.DMA,
        ],
        **_SC_PARAMS,
    )
    def k(sflat, sscore, psp_h, ps_h, o_pcs, o_ss,
          ipsp, b_pcs, ips, b_ss, sem):
        wid = lax.axis_index("s") * 2 + lax.axis_index("c")
        b3 = wid * _BS3
        b1 = wid * _BS
        pltpu.sync_copy(psp_h.at[pl.ds(b3, _BS3)], ipsp)
        pltpu.sync_copy(ps_h.at[pl.ds(b1, _BS)], ips)
        pltpu.async_copy(sflat.at[ipsp], b_pcs, sem).wait()
        pltpu.sync_copy(b_pcs, o_pcs.at[pl.ds(b3, _BS3)])
        pltpu.async_copy(sscore.at[ips], b_ss, sem).wait()
        pltpu.sync_copy(b_ss, o_ss.at[pl.ds(b1, _BS)])

    pcs_f, ss = k(src_flat, src_scores, psp, ps)
    return jnp.reshape(pcs_f, (3, MAX_ANCHOR)), ss


def _sc_gather_dstside(dst_flat):
    pdp = jnp.asarray(_IDXP_DST)

    @functools.partial(
        pl.kernel,
        out_type=jax.ShapeDtypeStruct((3 * MAX_DST,), jnp.float32),
        scratch_types=[
            pltpu.VMEM((_BD3,), jnp.int32),
            pltpu.VMEM((_BD3,), jnp.float32),
            pltpu.SemaphoreType.DMA,
        ],
        **_SC_PARAMS,
    )
    def k(dflat, pdp_h, o_pcd, ipdp, b_pcd, sem):
        wid = lax.axis_index("s") * 2 + lax.axis_index("c")
        b3 = wid * _BD3
        pltpu.sync_copy(pdp_h.at[pl.ds(b3, _BD3)], ipdp)
        pltpu.async_copy(dflat.at[ipdp], b_pcd, sem).wait()
        pltpu.sync_copy(b_pcd, o_pcd.at[pl.ds(b3, _BD3)])

    pcd_f = k(dst_flat, pdp)
    return jnp.reshape(pcd_f, (3, MAX_DST))


def _sc_gather_dstdesc(dst_desc, dst_scores, nn):
    pd = jnp.asarray(_PERM_DST)

    @functools.partial(
        pl.kernel,
        out_type=[
            jax.ShapeDtypeStruct((MAX_DST, 64), jnp.float32),
            jax.ShapeDtypeStruct((MAX_ANCHOR, 64), jnp.float32),
            jax.ShapeDtypeStruct((MAX_ANCHOR,), jnp.float32),
        ],
        scratch_types=[
            pltpu.VMEM((_BD,), jnp.int32),
            pltpu.VMEM((_BD, 64), jnp.float32),
            pltpu.VMEM((_BS,), jnp.int32),
            pltpu.VMEM((_BS, 64), jnp.float32),
            pltpu.VMEM((_BS,), jnp.float32),
            pltpu.SemaphoreType.DMA,
        ],
        **_SC_PARAMS,
    )
    def k(ddesc, dscore, pd_h, nn_h, o_dds, o_pos, o_ns,
          ipd, b_dds, inn, b_pos, b_ns, sem):
        wid = lax.axis_index("s") * 2 + lax.axis_index("c")
        b2 = wid * _BD
        b1 = wid * _BS
        pltpu.sync_copy(pd_h.at[pl.ds(b2, _BD)], ipd)
        pltpu.sync_copy(nn_h.at[pl.ds(b1, _BS)], inn)
        pltpu.async_copy(ddesc.at[ipd], b_dds, sem).wait()
        pltpu.sync_copy(b_dds, o_dds.at[pl.ds(b2, _BD)])
        pltpu.async_copy(ddesc.at[inn], b_pos, sem).wait()
        pltpu.sync_copy(b_pos, o_pos.at[pl.ds(b1, _BS)])
        pltpu.async_copy(dscore.at[inn], b_ns, sem).wait()
        pltpu.sync_copy(b_ns, o_ns.at[pl.ds(b1, _BS)])

    return k(dst_desc, dst_scores, pd, nn)


def _nn_kernel(a_ref, b_ref, mind_ref, idx_ref):
    j = pl.program_id(0)
    a = a_ref[...]
    b = b_ref[...]
    sa = jnp.sum(a * a, axis=0)
    sb = jnp.sum(b * b, axis=1)
    prod = lax.dot_general(a * -2.0, b, (((0,), (1,)), ((), ())),
                           preferred_element_type=jnp.float32)
    d2 = (sa[:, None] + sb[None, :]) + prod
    d = jnp.sqrt(jnp.maximum(d2, 1e-12))
    col = j * _NN_BLK + lax.broadcasted_iota(jnp.int32, d.shape, 1)
    blk_min = jnp.min(d, axis=1)
    blk_idx = jnp.min(jnp.where(d == blk_min[:, None], col, _N), axis=1)

    @pl.when(j == 0)
    def _():
        mind_ref[...] = blk_min
        idx_ref[...] = blk_idx

    @pl.when(j > 0)
    def _():
        prev = mind_ref[...]
        better = blk_min < prev
        mind_ref[...] = jnp.where(better, blk_min, prev)
        idx_ref[...] = jnp.where(better, blk_idx, idx_ref[...])


def _nn_search(pc_srcT, dst_xyz):
    grid = _N // _NN_BLK
    mind, idx = pl.pallas_call(
        _nn_kernel,
        grid=(grid,),
        in_specs=[
            pl.BlockSpec((3, MAX_ANCHOR), lambda j: (0, 0)),
            pl.BlockSpec((_NN_BLK, 3), lambda j: (j, 0)),
        ],
        out_specs=[
            pl.BlockSpec((MAX_ANCHOR,), lambda j: (0,)),
            pl.BlockSpec((MAX_ANCHOR,), lambda j: (0,)),
        ],
        out_shape=[
            jax.ShapeDtypeStruct((MAX_ANCHOR,), jnp.float32),
            jax.ShapeDtypeStruct((MAX_ANCHOR,), jnp.int32),
        ],
    )(pc_srcT, dst_xyz)
    return mind, idx


def _negmin_kernel(a_ref, ad_ref, b_ref, bd_ref, negmin_ref):
    j = pl.program_id(0)
    a = a_ref[...]
    b = b_ref[...]
    ad = ad_ref[...]
    bd = bd_ref[...]
    sa = jnp.sum(a * a, axis=0)
    sad = jnp.sum(ad * ad, axis=1)
    sb = jnp.sum(b * b, axis=0)
    sbd = jnp.sum(bd * bd, axis=1)

    prod_x = lax.dot_general(a * -2.0, b, (((0,), (0,)), ((), ())),
                             preferred_element_type=jnp.float32)
    dist2 = (sa[:, None] + sb[None, :]) + prod_x
    dist = jnp.sqrt(jnp.maximum(dist2, 1e-12))

    prod_d = lax.dot_general(ad * -2.0, bd, (((1,), (1,)), ((), ())),
                             preferred_element_type=jnp.float32)
    desc2 = (sad[:, None] + sbd[None, :]) + prod_d
    desc = jnp.sqrt(jnp.maximum(desc2, 1e-12))
    desc = desc + jnp.where(dist < NEG_RADIUS, 1e10, 0.0)
    blk_min = jnp.min(desc, axis=1)

    @pl.when(j == 0)
    def _():
        negmin_ref[...] = blk_min

    @pl.when(j > 0)
    def _():
        negmin_ref[...] = jnp.minimum(negmin_ref[...], blk_min)


def _negmin(pc_srcT, anc_desc, pc_dstT, desc_dst_sub):
    grid = MAX_DST // _NEG_BLK
    return pl.pallas_call(
        _negmin_kernel,
        grid=(grid,),
        in_specs=[
            pl.BlockSpec((3, MAX_ANCHOR), lambda j: (0, 0)),
            pl.BlockSpec((MAX_ANCHOR, 64), lambda j: (0, 0)),
            pl.BlockSpec((3, _NEG_BLK), lambda j: (0, j)),
            pl.BlockSpec((_NEG_BLK, 64), lambda j: (j, 0)),
        ],
        out_specs=pl.BlockSpec((MAX_ANCHOR,), lambda j: (0,)),
        out_shape=jax.ShapeDtypeStruct((MAX_ANCHOR,), jnp.float32),
    )(pc_srcT, anc_desc, pc_dstT, desc_dst_sub)


def _final_kernel(negmin_ref, ad_ref, pos_ref, ss_ref, ns_ref, nnd_ref,
                  out_ref):
    negative_min = negmin_ref[...]
    ad = ad_ref[...]
    pos = pos_ref[...]
    diff = ad - pos
    positive_max = jnp.sqrt(jnp.sum(diff * diff, axis=1) + 1e-12)
    p_n_diff = positive_max - negative_min
    nnd = nnd_ref[...]
    maskf = (nnd < POS_RADIUS).astype(jnp.float32)
    count = jnp.sum(maskf)
    sel_sigma = (ss_ref[...] + ns_ref[...]) * 0.5
    desc_loss = jnp.sum(jnp.maximum(p_n_diff + TRIPLET_MARGIN, 0.0) * maskf)
    score_loss = jnp.sum(sel_sigma * p_n_diff * maskf)
    loss = (desc_loss + score_loss) / count
    loss = jnp.where(count < float(MAX_ANCHOR // 2), 0.0, loss)
    out_ref[...] = loss.reshape(1, 1)


def _final_loss(negmin, anc_desc, pos_desc, s_src, s_nn, nn_d):
    out = pl.pallas_call(
        _final_kernel,
        out_shape=jax.ShapeDtypeStruct((1, 1), jnp.float32),
    )(negmin, anc_desc, pos_desc, s_src, s_nn, nn_d)
    return out[0, 0]


def kernel(src_xyz, src_desc, src_scores, dst_xyz, dst_desc, dst_scores, epoch):
    src_flat = jnp.reshape(src_xyz, (-1,))
    dst_flat = jnp.reshape(dst_xyz, (-1,))
    pc_srcT, s_src = _sc_gather_srcside(src_flat, src_scores)
    anc_desc = src_desc[_PERM_SRC]
    pc_dstT = _sc_gather_dstside(dst_flat)

    nn_d, nn = _nn_search(pc_srcT, dst_xyz)

    desc_dst_sub, pos_desc, s_nn = _sc_gather_dstdesc(dst_desc, dst_scores, nn)

    negmin = _negmin(pc_srcT, anc_desc, pc_dstT, desc_dst_sub)
    loss = _final_loss(negmin, anc_desc, pos_desc, s_src, s_nn, nn_d)
    out = jnp.where(jnp.asarray(epoch) <= VOTING_START, 0.0, loss)
    return out.astype(jnp.float32)


# speedup vs baseline: 1.0611x; 1.0252x over previous
import functools

import numpy as np
import jax
import jax.numpy as jnp
from jax import lax
from jax.experimental import pallas as pl
from jax.experimental.pallas import tpu as pltpu
from jax.experimental.pallas import tpu_sc as plsc

POS_RADIUS = 0.1
NEG_RADIUS = 0.2
TRIPLET_MARGIN = 1.0
MAX_ANCHOR = 1024
MAX_DST = 8192
VOTING_START = 0

_N = 20000
_rng = np.random.RandomState(0)
_PERM_SRC = np.ascontiguousarray(_rng.permutation(_N)[:MAX_ANCHOR].astype(np.int32))
_PERM_DST = np.ascontiguousarray(_rng.permutation(_N)[:MAX_DST].astype(np.int32))

_NN_BLK = 4000
_NEG_BLK = 4096

_NW = 32
_BS = MAX_ANCHOR // _NW
_BD = MAX_DST // _NW
_BS3 = _BS * 3
_BD3 = _BD * 3
_IDXP_SRC = np.ascontiguousarray(
    (_PERM_SRC[None, :] * 3 + np.arange(3)[:, None]).reshape(-1).astype(np.int32))
_IDXP_DST = np.ascontiguousarray(
    (_PERM_DST[None, :] * 3 + np.arange(3)[:, None]).reshape(-1).astype(np.int32))

_SC_MESH = dict(core_axis_name="c", subcore_axis_name="s")
_SC_PARAMS = dict(
    mesh=plsc.VectorSubcoreMesh(**_SC_MESH),
    compiler_params=pltpu.CompilerParams(use_tc_tiling_on_sc=False),
)


def _sc_gather_srcside(src_flat, src_scores):
    psp = jnp.asarray(_IDXP_SRC)
    ps = jnp.asarray(_PERM_SRC)

    @functools.partial(
        pl.kernel,
        out_type=[
            jax.ShapeDtypeStruct((3 * MAX_ANCHOR,), jnp.float32),
            jax.ShapeDtypeStruct((MAX_ANCHOR,), jnp.float32),
        ],
        scratch_types=[
            pltpu.VMEM((_BS3,), jnp.int32),
            pltpu.VMEM((_BS3,), jnp.float32),
            pltpu.VMEM((_BS,), jnp.int32),
            pltpu.VMEM((_BS,), jnp.float32),
            pltpu.SemaphoreType.DMA,
        ],
        **_SC_PARAMS,
    )
    def k(sflat, sscore, psp_h, ps_h, o_pcs, o_ss,
          ipsp, b_pcs, ips, b_ss, sem):
        wid = lax.axis_index("s") * 2 + lax.axis_index("c")
        b3 = wid * _BS3
        b1 = wid * _BS
        pltpu.sync_copy(psp_h.at[pl.ds(b3, _BS3)], ipsp)
        pltpu.sync_copy(ps_h.at[pl.ds(b1, _BS)], ips)
        pltpu.async_copy(sflat.at[ipsp], b_pcs, sem).wait()
        pltpu.sync_copy(b_pcs, o_pcs.at[pl.ds(b3, _BS3)])
        pltpu.async_copy(sscore.at[ips], b_ss, sem).wait()
        pltpu.sync_copy(b_ss, o_ss.at[pl.ds(b1, _BS)])

    pcs_f, ss = k(src_flat, src_scores, psp, ps)
    return jnp.reshape(pcs_f, (3, MAX_ANCHOR)), ss


def _sc_gather_dstside(dst_flat):
    pdp = jnp.asarray(_IDXP_DST)

    @functools.partial(
        pl.kernel,
        out_type=jax.ShapeDtypeStruct((3 * MAX_DST,), jnp.float32),
        scratch_types=[
            pltpu.VMEM((_BD3,), jnp.int32),
            pltpu.VMEM((_BD3,), jnp.float32),
            pltpu.SemaphoreType.DMA,
        ],
        **_SC_PARAMS,
    )
    def k(dflat, pdp_h, o_pcd, ipdp, b_pcd, sem):
        wid = lax.axis_index("s") * 2 + lax.axis_index("c")
        b3 = wid * _BD3
        pltpu.sync_copy(pdp_h.at[pl.ds(b3, _BD3)], ipdp)
        pltpu.async_copy(dflat.at[ipdp], b_pcd, sem).wait()
        pltpu.sync_copy(b_pcd, o_pcd.at[pl.ds(b3, _BD3)])

    pcd_f = k(dst_flat, pdp)
    return jnp.reshape(pcd_f, (3, MAX_DST))


def _sc_gather_dstdesc(dst_desc, dst_scores, nn):
    pd = jnp.asarray(_PERM_DST)

    @functools.partial(
        pl.kernel,
        out_type=[
            jax.ShapeDtypeStruct((MAX_DST, 64), jnp.float32),
            jax.ShapeDtypeStruct((MAX_ANCHOR, 64), jnp.float32),
            jax.ShapeDtypeStruct((MAX_ANCHOR,), jnp.float32),
        ],
        scratch_types=[
            pltpu.VMEM((_BD,), jnp.int32),
            pltpu.VMEM((_BD, 64), jnp.float32),
            pltpu.VMEM((_BS,), jnp.int32),
            pltpu.VMEM((_BS, 64), jnp.float32),
            pltpu.VMEM((_BS,), jnp.float32),
            pltpu.SemaphoreType.DMA,
        ],
        **_SC_PARAMS,
    )
    def k(ddesc, dscore, pd_h, nn_h, o_dds, o_pos, o_ns,
          ipd, b_dds, inn, b_pos, b_ns, sem):
        wid = lax.axis_index("s") * 2 + lax.axis_index("c")
        b2 = wid * _BD
        b1 = wid * _BS
        pltpu.sync_copy(pd_h.at[pl.ds(b2, _BD)], ipd)
        pltpu.sync_copy(nn_h.at[pl.ds(b1, _BS)], inn)
        pltpu.async_copy(ddesc.at[ipd], b_dds, sem).wait()
        pltpu.sync_copy(b_dds, o_dds.at[pl.ds(b2, _BD)])
        pltpu.async_copy(ddesc.at[inn], b_pos, sem).wait()
        pltpu.sync_copy(b_pos, o_pos.at[pl.ds(b1, _BS)])
        pltpu.async_copy(dscore.at[inn], b_ns, sem).wait()
        pltpu.sync_copy(b_ns, o_ns.at[pl.ds(b1, _BS)])

    return k(dst_desc, dst_scores, pd, nn)


def _nn_kernel(a_ref, b_ref, mind_ref, idx_ref):
    j = pl.program_id(0)
    a = a_ref[...]
    b = b_ref[...]
    sa = jnp.sum(a * a, axis=0)
    sb = jnp.sum(b * b, axis=1)
    prod = lax.dot_general(a * -2.0, b, (((0,), (1,)), ((), ())),
                           preferred_element_type=jnp.float32)
    d2 = (sa[:, None] + sb[None, :]) + prod
    d = jnp.sqrt(jnp.maximum(d2, 1e-12))
    col = j * _NN_BLK + lax.broadcasted_iota(jnp.int32, d.shape, 1)
    blk_min = jnp.min(d, axis=1)
    blk_idx = jnp.min(jnp.where(d == blk_min[:, None], col, _N), axis=1)

    @pl.when(j == 0)
    def _():
        mind_ref[...] = blk_min
        idx_ref[...] = blk_idx

    @pl.when(j > 0)
    def _():
        prev = mind_ref[...]
        better = blk_min < prev
        mind_ref[...] = jnp.where(better, blk_min, prev)
        idx_ref[...] = jnp.where(better, blk_idx, idx_ref[...])


def _nn_search(pc_srcT, dst_xyz):
    grid = _N // _NN_BLK
    mind, idx = pl.pallas_call(
        _nn_kernel,
        grid=(grid,),
        in_specs=[
            pl.BlockSpec((3, MAX_ANCHOR), lambda j: (0, 0)),
            pl.BlockSpec((_NN_BLK, 3), lambda j: (j, 0)),
        ],
        out_specs=[
            pl.BlockSpec((MAX_ANCHOR,), lambda j: (0,)),
            pl.BlockSpec((MAX_ANCHOR,), lambda j: (0,)),
        ],
        out_shape=[
            jax.ShapeDtypeStruct((MAX_ANCHOR,), jnp.float32),
            jax.ShapeDtypeStruct((MAX_ANCHOR,), jnp.int32),
        ],
    )(pc_srcT, dst_xyz)
    return mind, idx


def _negmin_kernel(a_ref, ad_ref, b_ref, bd_ref, negmin_ref):
    j = pl.program_id(0)
    a = a_ref[...]
    b = b_ref[...]
    ad = ad_ref[...]
    bd = bd_ref[...]
    sa = jnp.sum(a * a, axis=0)
    sad = jnp.sum(ad * ad, axis=1)
    sb = jnp.sum(b * b, axis=0)
    sbd = jnp.sum(bd * bd, axis=1)

    prod_x = lax.dot_general(a * -2.0, b, (((0,), (0,)), ((), ())),
                             preferred_element_type=jnp.float32)
    dist2 = (sa[:, None] + sb[None, :]) + prod_x
    dist = jnp.sqrt(jnp.maximum(dist2, 1e-12))

    prod_d = lax.dot_general(ad * -2.0, bd, (((1,), (1,)), ((), ())),
                             preferred_element_type=jnp.float32)
    desc2 = (sad[:, None] + sbd[None, :]) + prod_d
    desc = jnp.sqrt(jnp.maximum(desc2, 1e-12))
    desc = desc + jnp.where(dist < NEG_RADIUS, 1e10, 0.0)
    blk_min = jnp.min(desc, axis=1)

    @pl.when(j == 0)
    def _():
        negmin_ref[...] = blk_min

    @pl.when(j > 0)
    def _():
        negmin_ref[...] = jnp.minimum(negmin_ref[...], blk_min)


def _negmin(pc_srcT, anc_desc, pc_dstT, desc_dst_sub):
    grid = MAX_DST // _NEG_BLK
    return pl.pallas_call(
        _negmin_kernel,
        grid=(grid,),
        in_specs=[
            pl.BlockSpec((3, MAX_ANCHOR), lambda j: (0, 0)),
            pl.BlockSpec((MAX_ANCHOR, 64), lambda j: (0, 0)),
            pl.BlockSpec((3, _NEG_BLK), lambda j: (0, j)),
            pl.BlockSpec((_NEG_BLK, 64), lambda j: (j, 0)),
        ],
        out_specs=pl.BlockSpec((MAX_ANCHOR,), lambda j: (0,)),
        out_shape=jax.ShapeDtypeStruct((MAX_ANCHOR,), jnp.float32),
    )(pc_srcT, anc_desc, pc_dstT, desc_dst_sub)


def _final_kernel(negmin_ref, ad_ref, pos_ref, ss_ref, ns_ref, nnd_ref,
                  out_ref):
    negative_min = negmin_ref[...]
    ad = ad_ref[...]
    pos = pos_ref[...]
    diff = ad - pos
    positive_max = jnp.sqrt(jnp.sum(diff * diff, axis=1) + 1e-12)
    p_n_diff = positive_max - negative_min
    nnd = nnd_ref[...]
    maskf = (nnd < POS_RADIUS).astype(jnp.float32)
    count = jnp.sum(maskf)
    sel_sigma = (ss_ref[...] + ns_ref[...]) * 0.5
    desc_loss = jnp.sum(jnp.maximum(p_n_diff + TRIPLET_MARGIN, 0.0) * maskf)
    score_loss = jnp.sum(sel_sigma * p_n_diff * maskf)
    loss = (desc_loss + score_loss) / count
    loss = jnp.where(count < float(MAX_ANCHOR // 2), 0.0, loss)
    out_ref[...] = loss.reshape(1, 1)


def _final_loss(negmin, anc_desc, pos_desc, s_src, s_nn, nn_d):
    out = pl.pallas_call(
        _final_kernel,
        out_shape=jax.ShapeDtypeStruct((1, 1), jnp.float32),
    )(negmin, anc_desc, pos_desc, s_src, s_nn, nn_d)
    return out[0, 0]


def kernel(src_xyz, src_desc, src_scores, dst_xyz, dst_desc, dst_scores, epoch):
    src_flat = jnp.reshape(src_xyz, (-1,))
    dst_flat = jnp.reshape(dst_xyz, (-1,))
    pc_srcT, s_src = _sc_gather_srcside(src_flat, src_scores)
    anc_desc = src_desc[_PERM_SRC]
    pc_dstT = _sc_gather_dstside(dst_flat)

    nn_d, nn = _nn_search(pc_srcT, dst_xyz)

    desc_dst_sub, pos_desc, s_nn = _sc_gather_dstdesc(dst_desc, dst_scores, nn)

    negmin = _negmin(pc_srcT, anc_desc, pc_dstT, desc_dst_sub)
    loss = _final_loss(negmin, anc_desc, pos_desc, s_src, s_nn, nn_d)
    out = jnp.where(jnp.asarray(epoch) <= VOTING_START, 0.0, loss)
    return out.astype(jnp.float32)


# speedup vs baseline: 1.0663x; 1.0049x over previous
import functools

import numpy as np
import jax
import jax.numpy as jnp
from jax import lax
from jax.experimental import pallas as pl
from jax.experimental.pallas import tpu as pltpu
from jax.experimental.pallas import tpu_sc as plsc

POS_RADIUS = 0.1
NEG_RADIUS = 0.2
TRIPLET_MARGIN = 1.0
MAX_ANCHOR = 1024
MAX_DST = 8192
VOTING_START = 0

_N = 20000
_rng = np.random.RandomState(0)
_PERM_SRC = np.ascontiguousarray(_rng.permutation(_N)[:MAX_ANCHOR].astype(np.int32))
_PERM_DST = np.ascontiguousarray(_rng.permutation(_N)[:MAX_DST].astype(np.int32))

_NN_BLK = 5000
_NEG_BLK = 4096

_NW = 32
_BS = MAX_ANCHOR // _NW
_BD = MAX_DST // _NW
_BS3 = _BS * 3
_BD3 = _BD * 3
_IDXP_SRC = np.ascontiguousarray(
    (_PERM_SRC[None, :] * 3 + np.arange(3)[:, None]).reshape(-1).astype(np.int32))
_IDXP_DST = np.ascontiguousarray(
    (_PERM_DST[None, :] * 3 + np.arange(3)[:, None]).reshape(-1).astype(np.int32))

_SC_MESH = dict(core_axis_name="c", subcore_axis_name="s")
_SC_PARAMS = dict(
    mesh=plsc.VectorSubcoreMesh(**_SC_MESH),
    compiler_params=pltpu.CompilerParams(use_tc_tiling_on_sc=False),
)


def _sc_gather_srcside(src_flat, src_scores):
    psp = jnp.asarray(_IDXP_SRC)
    ps = jnp.asarray(_PERM_SRC)

    @functools.partial(
        pl.kernel,
        out_type=[
            jax.ShapeDtypeStruct((3 * MAX_ANCHOR,), jnp.float32),
            jax.ShapeDtypeStruct((MAX_ANCHOR,), jnp.float32),
        ],
        scratch_types=[
            pltpu.VMEM((_BS3,), jnp.int32),
            pltpu.VMEM((_BS3,), jnp.float32),
            pltpu.VMEM((_BS,), jnp.int32),
            pltpu.VMEM((_BS,), jnp.float32),
            pltpu.SemaphoreType.DMA,
        ],
        **_SC_PARAMS,
    )
    def k(sflat, sscore, psp_h, ps_h, o_pcs, o_ss,
          ipsp, b_pcs, ips, b_ss, sem):
        wid = lax.axis_index("s") * 2 + lax.axis_index("c")
        b3 = wid * _BS3
        b1 = wid * _BS
        pltpu.sync_copy(psp_h.at[pl.ds(b3, _BS3)], ipsp)
        pltpu.sync_copy(ps_h.at[pl.ds(b1, _BS)], ips)
        pltpu.async_copy(sflat.at[ipsp], b_pcs, sem).wait()
        pltpu.sync_copy(b_pcs, o_pcs.at[pl.ds(b3, _BS3)])
        pltpu.async_copy(sscore.at[ips], b_ss, sem).wait()
        pltpu.sync_copy(b_ss, o_ss.at[pl.ds(b1, _BS)])

    pcs_f, ss = k(src_flat, src_scores, psp, ps)
    return jnp.reshape(pcs_f, (3, MAX_ANCHOR)), ss


def _sc_gather_dstside(dst_flat):
    pdp = jnp.asarray(_IDXP_DST)

    @functools.partial(
        pl.kernel,
        out_type=jax.ShapeDtypeStruct((3 * MAX_DST,), jnp.float32),
        scratch_types=[
            pltpu.VMEM((_BD3,), jnp.int32),
            pltpu.VMEM((_BD3,), jnp.float32),
            pltpu.SemaphoreType.DMA,
        ],
        **_SC_PARAMS,
    )
    def k(dflat, pdp_h, o_pcd, ipdp, b_pcd, sem):
        wid = lax.axis_index("s") * 2 + lax.axis_index("c")
        b3 = wid * _BD3
        pltpu.sync_copy(pdp_h.at[pl.ds(b3, _BD3)], ipdp)
        pltpu.async_copy(dflat.at[ipdp], b_pcd, sem).wait()
        pltpu.sync_copy(b_pcd, o_pcd.at[pl.ds(b3, _BD3)])

    pcd_f = k(dst_flat, pdp)
    return jnp.reshape(pcd_f, (3, MAX_DST))


def _sc_gather_dstdesc(dst_desc, dst_scores, nn):
    pd = jnp.asarray(_PERM_DST)

    @functools.partial(
        pl.kernel,
        out_type=[
            jax.ShapeDtypeStruct((MAX_DST, 64), jnp.float32),
            jax.ShapeDtypeStruct((MAX_ANCHOR, 64), jnp.float32),
            jax.ShapeDtypeStruct((MAX_ANCHOR,), jnp.float32),
        ],
        scratch_types=[
            pltpu.VMEM((_BD,), jnp.int32),
            pltpu.VMEM((_BD, 64), jnp.float32),
            pltpu.VMEM((_BS,), jnp.int32),
            pltpu.VMEM((_BS, 64), jnp.float32),
            pltpu.VMEM((_BS,), jnp.float32),
            pltpu.SemaphoreType.DMA,
        ],
        **_SC_PARAMS,
    )
    def k(ddesc, dscore, pd_h, nn_h, o_dds, o_pos, o_ns,
          ipd, b_dds, inn, b_pos, b_ns, sem):
        wid = lax.axis_index("s") * 2 + lax.axis_index("c")
        b2 = wid * _BD
        b1 = wid * _BS
        pltpu.sync_copy(pd_h.at[pl.ds(b2, _BD)], ipd)
        pltpu.sync_copy(nn_h.at[pl.ds(b1, _BS)], inn)
        pltpu.async_copy(ddesc.at[ipd], b_dds, sem).wait()
        pltpu.sync_copy(b_dds, o_dds.at[pl.ds(b2, _BD)])
        pltpu.async_copy(ddesc.at[inn], b_pos, sem).wait()
        pltpu.sync_copy(b_pos, o_pos.at[pl.ds(b1, _BS)])
        pltpu.async_copy(dscore.at[inn], b_ns, sem).wait()
        pltpu.sync_copy(b_ns, o_ns.at[pl.ds(b1, _BS)])

    return k(dst_desc, dst_scores, pd, nn)


def _nn_kernel(a_ref, b_ref, mind_ref, idx_ref):
    j = pl.program_id(0)
    a = a_ref[...]
    b = b_ref[...]
    sa = jnp.sum(a * a, axis=0)
    sb = jnp.sum(b * b, axis=1)
    prod = lax.dot_general(a * -2.0, b, (((0,), (1,)), ((), ())),
                           preferred_element_type=jnp.float32)
    d2 = (sa[:, None] + sb[None, :]) + prod
    d = jnp.sqrt(jnp.maximum(d2, 1e-12))
    col = j * _NN_BLK + lax.broadcasted_iota(jnp.int32, d.shape, 1)
    blk_min = jnp.min(d, axis=1)
    blk_idx = jnp.min(jnp.where(d == blk_min[:, None], col, _N), axis=1)

    @pl.when(j == 0)
    def _():
        mind_ref[...] = blk_min
        idx_ref[...] = blk_idx

    @pl.when(j > 0)
    def _():
        prev = mind_ref[...]
        better = blk_min < prev
        mind_ref[...] = jnp.where(better, blk_min, prev)
        idx_ref[...] = jnp.where(better, blk_idx, idx_ref[...])


def _nn_search(pc_srcT, dst_xyz):
    grid = _N // _NN_BLK
    mind, idx = pl.pallas_call(
        _nn_kernel,
        grid=(grid,),
        in_specs=[
            pl.BlockSpec((3, MAX_ANCHOR), lambda j: (0, 0)),
            pl.BlockSpec((_NN_BLK, 3), lambda j: (j, 0)),
        ],
        out_specs=[
            pl.BlockSpec((MAX_ANCHOR,), lambda j: (0,)),
            pl.BlockSpec((MAX_ANCHOR,), lambda j: (0,)),
        ],
        out_shape=[
            jax.ShapeDtypeStruct((MAX_ANCHOR,), jnp.float32),
            jax.ShapeDtypeStruct((MAX_ANCHOR,), jnp.int32),
        ],
    )(pc_srcT, dst_xyz)
    return mind, idx


def _negmin_kernel(a_ref, ad_ref, b_ref, bd_ref, negmin_ref):
    j = pl.program_id(0)
    a = a_ref[...]
    b = b_ref[...]
    ad = ad_ref[...]
    bd = bd_ref[...]
    sa = jnp.sum(a * a, axis=0)
    sad = jnp.sum(ad * ad, axis=1)
    sb = jnp.sum(b * b, axis=0)
    sbd = jnp.sum(bd * bd, axis=1)

    prod_x = lax.dot_general(a * -2.0, b, (((0,), (0,)), ((), ())),
                             preferred_element_type=jnp.float32)
    dist2 = (sa[:, None] + sb[None, :]) + prod_x
    dist = jnp.sqrt(jnp.maximum(dist2, 1e-12))

    prod_d = lax.dot_general(ad * -2.0, bd, (((1,), (1,)), ((), ())),
                             preferred_element_type=jnp.float32)
    desc2 = (sad[:, None] + sbd[None, :]) + prod_d
    desc = jnp.sqrt(jnp.maximum(desc2, 1e-12))
    desc = desc + jnp.where(dist < NEG_RADIUS, 1e10, 0.0)
    blk_min = jnp.min(desc, axis=1)

    @pl.when(j == 0)
    def _():
        negmin_ref[...] = blk_min

    @pl.when(j > 0)
    def _():
        negmin_ref[...] = jnp.minimum(negmin_ref[...], blk_min)


def _negmin(pc_srcT, anc_desc, pc_dstT, desc_dst_sub):
    grid = MAX_DST // _NEG_BLK
    return pl.pallas_call(
        _negmin_kernel,
        grid=(grid,),
        in_specs=[
            pl.BlockSpec((3, MAX_ANCHOR), lambda j: (0, 0)),
            pl.BlockSpec((MAX_ANCHOR, 64), lambda j: (0, 0)),
            pl.BlockSpec((3, _NEG_BLK), lambda j: (0, j)),
            pl.BlockSpec((_NEG_BLK, 64), lambda j: (j, 0)),
        ],
        out_specs=pl.BlockSpec((MAX_ANCHOR,), lambda j: (0,)),
        out_shape=jax.ShapeDtypeStruct((MAX_ANCHOR,), jnp.float32),
    )(pc_srcT, anc_desc, pc_dstT, desc_dst_sub)


def _final_kernel(negmin_ref, ad_ref, pos_ref, ss_ref, ns_ref, nnd_ref,
                  out_ref):
    negative_min = negmin_ref[...]
    ad = ad_ref[...]
    pos = pos_ref[...]
    diff = ad - pos
    positive_max = jnp.sqrt(jnp.sum(diff * diff, axis=1) + 1e-12)
    p_n_diff = positive_max - negative_min
    nnd = nnd_ref[...]
    maskf = (nnd < POS_RADIUS).astype(jnp.float32)
    count = jnp.sum(maskf)
    sel_sigma = (ss_ref[...] + ns_ref[...]) * 0.5
    desc_loss = jnp.sum(jnp.maximum(p_n_diff + TRIPLET_MARGIN, 0.0) * maskf)
    score_loss = jnp.sum(sel_sigma * p_n_diff * maskf)
    loss = (desc_loss + score_loss) / count
    loss = jnp.where(count < float(MAX_ANCHOR // 2), 0.0, loss)
    out_ref[...] = loss.reshape(1, 1)


def _final_loss(negmin, anc_desc, pos_desc, s_src, s_nn, nn_d):
    out = pl.pallas_call(
        _final_kernel,
        out_shape=jax.ShapeDtypeStruct((1, 1), jnp.float32),
    )(negmin, anc_desc, pos_desc, s_src, s_nn, nn_d)
    return out[0, 0]


def kernel(src_xyz, src_desc, src_scores, dst_xyz, dst_desc, dst_scores, epoch):
    src_flat = jnp.reshape(src_xyz, (-1,))
    dst_flat = jnp.reshape(dst_xyz, (-1,))
    pc_srcT, s_src = _sc_gather_srcside(src_flat, src_scores)
    anc_desc = src_desc[_PERM_SRC]
    pc_dstT = _sc_gather_dstside(dst_flat)

    nn_d, nn = _nn_search(pc_srcT, dst_xyz)

    desc_dst_sub, pos_desc, s_nn = _sc_gather_dstdesc(dst_desc, dst_scores, nn)

    negmin = _negmin(pc_srcT, anc_desc, pc_dstT, desc_dst_sub)
    loss = _final_loss(negmin, anc_desc, pos_desc, s_src, s_nn, nn_d)
    out = jnp.where(jnp.asarray(epoch) <= VOTING_START, 0.0, loss)
    return out.astype(jnp.float32)


# speedup vs baseline: 1.0766x; 1.0096x over previous
import functools

import numpy as np
import jax
import jax.numpy as jnp
from jax import lax
from jax.experimental import pallas as pl
from jax.experimental.pallas import tpu as pltpu
from jax.experimental.pallas import tpu_sc as plsc

POS_RADIUS = 0.1
NEG_RADIUS = 0.2
TRIPLET_MARGIN = 1.0
MAX_ANCHOR = 1024
MAX_DST = 8192
VOTING_START = 0

_N = 20000
_rng = np.random.RandomState(0)
_PERM_SRC = np.ascontiguousarray(_rng.permutation(_N)[:MAX_ANCHOR].astype(np.int32))
_PERM_DST = np.ascontiguousarray(_rng.permutation(_N)[:MAX_DST].astype(np.int32))

_NN_BLK = 5000
_NEG_BLK = 4096

_NW = 32
_BS = MAX_ANCHOR // _NW
_BD = MAX_DST // _NW
_BS3 = _BS * 3
_BD3 = _BD * 3
_IDXP_SRC = np.ascontiguousarray(
    (_PERM_SRC[None, :] * 3 + np.arange(3)[:, None]).reshape(-1).astype(np.int32))
_IDXP_DST = np.ascontiguousarray(
    (_PERM_DST[None, :] * 3 + np.arange(3)[:, None]).reshape(-1).astype(np.int32))

_SC_MESH = dict(core_axis_name="c", subcore_axis_name="s")
_SC_PARAMS = dict(
    mesh=plsc.VectorSubcoreMesh(**_SC_MESH),
    compiler_params=pltpu.CompilerParams(use_tc_tiling_on_sc=False),
)


def _sc_gather_srcside(src_flat, src_scores):
    psp = jnp.asarray(_IDXP_SRC)
    ps = jnp.asarray(_PERM_SRC)

    @functools.partial(
        pl.kernel,
        out_type=[
            jax.ShapeDtypeStruct((3 * MAX_ANCHOR,), jnp.float32),
            jax.ShapeDtypeStruct((MAX_ANCHOR,), jnp.float32),
        ],
        scratch_types=[
            pltpu.VMEM((_BS3,), jnp.int32),
            pltpu.VMEM((_BS3,), jnp.float32),
            pltpu.VMEM((_BS,), jnp.int32),
            pltpu.VMEM((_BS,), jnp.float32),
            pltpu.SemaphoreType.DMA,
        ],
        **_SC_PARAMS,
    )
    def k(sflat, sscore, psp_h, ps_h, o_pcs, o_ss,
          ipsp, b_pcs, ips, b_ss, sem):
        wid = lax.axis_index("s") * 2 + lax.axis_index("c")
        b3 = wid * _BS3
        b1 = wid * _BS
        pltpu.sync_copy(psp_h.at[pl.ds(b3, _BS3)], ipsp)
        pltpu.sync_copy(ps_h.at[pl.ds(b1, _BS)], ips)
        pltpu.async_copy(sflat.at[ipsp], b_pcs, sem).wait()
        pltpu.sync_copy(b_pcs, o_pcs.at[pl.ds(b3, _BS3)])
        pltpu.async_copy(sscore.at[ips], b_ss, sem).wait()
        pltpu.sync_copy(b_ss, o_ss.at[pl.ds(b1, _BS)])

    pcs_f, ss = k(src_flat, src_scores, psp, ps)
    return jnp.reshape(pcs_f, (3, MAX_ANCHOR)), ss


def _sc_gather_dstside(dst_flat):
    pdp = jnp.asarray(_IDXP_DST)

    @functools.partial(
        pl.kernel,
        out_type=jax.ShapeDtypeStruct((3 * MAX_DST,), jnp.float32),
        scratch_types=[
            pltpu.VMEM((_BD3,), jnp.int32),
            pltpu.VMEM((_BD3,), jnp.float32),
            pltpu.SemaphoreType.DMA,
        ],
        **_SC_PARAMS,
    )
    def k(dflat, pdp_h, o_pcd, ipdp, b_pcd, sem):
        wid = lax.axis_index("s") * 2 + lax.axis_index("c")
        b3 = wid * _BD3
        pltpu.sync_copy(pdp_h.at[pl.ds(b3, _BD3)], ipdp)
        pltpu.async_copy(dflat.at[ipdp], b_pcd, sem).wait()
        pltpu.sync_copy(b_pcd, o_pcd.at[pl.ds(b3, _BD3)])

    pcd_f = k(dst_flat, pdp)
    return jnp.reshape(pcd_f, (3, MAX_DST))


def _sc_gather_dstdesc(dst_desc, dst_scores, nn):
    pd = jnp.asarray(_PERM_DST)

    @functools.partial(
        pl.kernel,
        out_type=[
            jax.ShapeDtypeStruct((MAX_DST, 64), jnp.float32),
            jax.ShapeDtypeStruct((MAX_ANCHOR, 64), jnp.float32),
            jax.ShapeDtypeStruct((MAX_ANCHOR,), jnp.float32),
        ],
        scratch_types=[
            pltpu.VMEM((_BD,), jnp.int32),
            pltpu.VMEM((_BD, 64), jnp.float32),
            pltpu.VMEM((_BS,), jnp.int32),
            pltpu.VMEM((_BS, 64), jnp.float32),
            pltpu.VMEM((_BS,), jnp.float32),
            pltpu.SemaphoreType.DMA,
        ],
        **_SC_PARAMS,
    )
    def k(ddesc, dscore, pd_h, nn_h, o_dds, o_pos, o_ns,
          ipd, b_dds, inn, b_pos, b_ns, sem):
        wid = lax.axis_index("s") * 2 + lax.axis_index("c")
        b2 = wid * _BD
        b1 = wid * _BS
        pltpu.sync_copy(pd_h.at[pl.ds(b2, _BD)], ipd)
        pltpu.sync_copy(nn_h.at[pl.ds(b1, _BS)], inn)
        pltpu.async_copy(ddesc.at[ipd], b_dds, sem).wait()
        pltpu.sync_copy(b_dds, o_dds.at[pl.ds(b2, _BD)])
        pltpu.async_copy(ddesc.at[inn], b_pos, sem).wait()
        pltpu.sync_copy(b_pos, o_pos.at[pl.ds(b1, _BS)])
        pltpu.async_copy(dscore.at[inn], b_ns, sem).wait()
        pltpu.sync_copy(b_ns, o_ns.at[pl.ds(b1, _BS)])

    return k(dst_desc, dst_scores, pd, nn)


def _nn_kernel(a_ref, b_ref, mind_ref, idx_ref):
    j = pl.program_id(0)
    a = a_ref[...]
    b = b_ref[...]
    sa = jnp.sum(a * a, axis=0)
    sb = jnp.sum(b * b, axis=1)
    prod = lax.dot_general(a * -2.0, b, (((0,), (1,)), ((), ())),
                           preferred_element_type=jnp.float32)
    d2 = (sa[:, None] + sb[None, :]) + prod
    d = jnp.sqrt(jnp.maximum(d2, 1e-12))
    col = j * _NN_BLK + lax.broadcasted_iota(jnp.int32, d.shape, 1)
    blk_min = jnp.min(d, axis=1)
    blk_idx = jnp.min(jnp.where(d == blk_min[:, None], col, _N), axis=1)

    @pl.when(j == 0)
    def _():
        mind_ref[...] = blk_min
        idx_ref[...] = blk_idx

    @pl.when(j > 0)
    def _():
        prev = mind_ref[...]
        better = blk_min < prev
        mind_ref[...] = jnp.where(better, blk_min, prev)
        idx_ref[...] = jnp.where(better, blk_idx, idx_ref[...])


def _nn_search(pc_srcT, dst_xyz):
    grid = _N // _NN_BLK
    mind, idx = pl.pallas_call(
        _nn_kernel,
        grid=(grid,),
        in_specs=[
            pl.BlockSpec((3, MAX_ANCHOR), lambda j: (0, 0)),
            pl.BlockSpec((_NN_BLK, 3), lambda j: (j, 0)),
        ],
        out_specs=[
            pl.BlockSpec((MAX_ANCHOR,), lambda j: (0,)),
            pl.BlockSpec((MAX_ANCHOR,), lambda j: (0,)),
        ],
        out_shape=[
            jax.ShapeDtypeStruct((MAX_ANCHOR,), jnp.float32),
            jax.ShapeDtypeStruct((MAX_ANCHOR,), jnp.int32),
        ],
    )(pc_srcT, dst_xyz)
    return mind, idx


def _negmin_kernel(a_ref, ad_ref, b_ref, bd_ref,
                   pos_ref, ss_ref, ns_ref, nnd_ref, out_ref, negmin_ref):
    j = pl.program_id(0)
    a = a_ref[...]
    b = b_ref[...]
    ad = ad_ref[...]
    bd = bd_ref[...]
    sa = jnp.sum(a * a, axis=0)
    sad = jnp.sum(ad * ad, axis=1)
    sb = jnp.sum(b * b, axis=0)
    sbd = jnp.sum(bd * bd, axis=1)

    prod_x = lax.dot_general(a * -2.0, b, (((0,), (0,)), ((), ())),
                             preferred_element_type=jnp.float32)
    dist2 = (sa[:, None] + sb[None, :]) + prod_x
    dist = jnp.sqrt(jnp.maximum(dist2, 1e-12))

    prod_d = lax.dot_general(ad * -2.0, bd, (((1,), (1,)), ((), ())),
                             preferred_element_type=jnp.float32)
    desc2 = (sad[:, None] + sbd[None, :]) + prod_d
    desc = jnp.sqrt(jnp.maximum(desc2, 1e-12))
    desc = desc + jnp.where(dist < NEG_RADIUS, 1e10, 0.0)
    blk_min = jnp.min(desc, axis=1)

    @pl.when(j == 0)
    def _():
        negmin_ref[...] = blk_min

    @pl.when(j > 0)
    def _():
        negmin_ref[...] = jnp.minimum(negmin_ref[...], blk_min)

    @pl.when(j == pl.num_programs(0) - 1)
    def _():
        negative_min = negmin_ref[...]
        pos = pos_ref[...]
        diff = ad - pos
        positive_max = jnp.sqrt(jnp.sum(diff * diff, axis=1) + 1e-12)
        p_n_diff = positive_max - negative_min
        nnd = nnd_ref[...]
        maskf = (nnd < POS_RADIUS).astype(jnp.float32)
        count = jnp.sum(maskf)
        sel_sigma = (ss_ref[...] + ns_ref[...]) * 0.5
        desc_loss = jnp.sum(jnp.maximum(p_n_diff + TRIPLET_MARGIN, 0.0) * maskf)
        score_loss = jnp.sum(sel_sigma * p_n_diff * maskf)
        loss = (desc_loss + score_loss) / count
        loss = jnp.where(count < float(MAX_ANCHOR // 2), 0.0, loss)
        out_ref[...] = loss.reshape(1, 1)


def _negmin_loss(pc_srcT, anc_desc, pc_dstT, desc_dst_sub,
                 pos_desc, s_src, s_nn, nn_d):
    grid = MAX_DST // _NEG_BLK
    out = pl.pallas_call(
        _negmin_kernel,
        grid=(grid,),
        in_specs=[
            pl.BlockSpec((3, MAX_ANCHOR), lambda j: (0, 0)),
            pl.BlockSpec((MAX_ANCHOR, 64), lambda j: (0, 0)),
            pl.BlockSpec((3, _NEG_BLK), lambda j: (0, j)),
            pl.BlockSpec((_NEG_BLK, 64), lambda j: (j, 0)),
            pl.BlockSpec((MAX_ANCHOR, 64), lambda j: (0, 0)),
            pl.BlockSpec((MAX_ANCHOR,), lambda j: (0,)),
            pl.BlockSpec((MAX_ANCHOR,), lambda j: (0,)),
            pl.BlockSpec((MAX_ANCHOR,), lambda j: (0,)),
        ],
        out_specs=pl.BlockSpec((1, 1), lambda j: (0, 0)),
        out_shape=jax.ShapeDtypeStruct((1, 1), jnp.float32),
        scratch_shapes=[pltpu.VMEM((MAX_ANCHOR,), jnp.float32)],
    )(pc_srcT, anc_desc, pc_dstT, desc_dst_sub,
      pos_desc, s_src, s_nn, nn_d)
    return out[0, 0]


def kernel(src_xyz, src_desc, src_scores, dst_xyz, dst_desc, dst_scores, epoch):
    src_flat = jnp.reshape(src_xyz, (-1,))
    dst_flat = jnp.reshape(dst_xyz, (-1,))
    pc_srcT, s_src = _sc_gather_srcside(src_flat, src_scores)
    anc_desc = src_desc[_PERM_SRC]
    pc_dstT = _sc_gather_dstside(dst_flat)

    nn_d, nn = _nn_search(pc_srcT, dst_xyz)

    desc_dst_sub, pos_desc, s_nn = _sc_gather_dstdesc(dst_desc, dst_scores, nn)

    loss = _negmin_loss(pc_srcT, anc_desc, pc_dstT, desc_dst_sub,
                        pos_desc, s_src, s_nn, nn_d)
    out = jnp.where(jnp.asarray(epoch) <= VOTING_START, 0.0, loss)
    return out.astype(jnp.float32)
